# Initial kernel scaffold; baseline (speedup 1.0000x reference)
import jax
import jax.numpy as jnp
from jax.experimental import pallas as pl


def kernel(hidden_states, boundary_mask, boundary_prob):
    raise NotImplementedError("write your pallas kernel here")


# speedup vs baseline: 34.1086x; 34.1086x over previous
import functools

import jax
import jax.numpy as jnp
from jax import lax
from jax.experimental import pallas as pl
from jax.experimental.pallas import tpu as pltpu
from jax.experimental.pallas import tpu_sc as plsc

B = 8
L = 4096
D = 1024
T = 256
K = L // T
NW = 32
ROWS_PER_W = (B * L) // NW
GCHUNK = 64
CLIP_LO = 1e-4
CLIP_HI = 1.0 - 1e-4


def _prep_body(mask_hbm, p_hbm, psort_hbm, idx_hbm,
               mask_v, p_v, psort_v, idx_v, sem):
    wid = lax.axis_index("s") * 2 + lax.axis_index("c")

    @pl.when(wid < B)
    def _():
        pltpu.async_copy(mask_hbm.at[wid], mask_v, sem).wait()
        pltpu.async_copy(p_hbm.at[wid], p_v, sem).wait()

        fill = jnp.full((16,), 0.5, jnp.float32)

        def body(j, cnt_vec):
            m16 = mask_v[pl.ds(j * 16, 16)]
            mb = m16 > 0
            p16 = p_v[pl.ds(j * 16, 16)]
            psort_v[pl.ds(j * 16, 16)] = fill
            cum = plsc.cumsum(m16)
            pbv = cum + cnt_vec - 1
            idx_v[pl.ds(j * 16, 16)] = pbv + wid * L
            plsc.store_scatter(psort_v, [pbv], p16, mask=mb)
            return cnt_vec + plsc.all_reduce_population_count(mb)

        lax.fori_loop(0, L // 16, body, jnp.zeros((16,), jnp.int32))

        pltpu.async_copy(psort_v, psort_hbm.at[wid], sem).wait()
        pltpu.async_copy(idx_v, idx_hbm.at[wid], sem).wait()


def _sc_prep(mask_i32, p_raw):
    mesh = plsc.VectorSubcoreMesh(core_axis_name="c", subcore_axis_name="s")
    fn = pl.kernel(
        _prep_body,
        mesh=mesh,
        out_type=(
            jax.ShapeDtypeStruct((B, L), jnp.float32),
            jax.ShapeDtypeStruct((B, L), jnp.int32),
        ),
        scratch_types=[
            pltpu.VMEM((L,), jnp.int32),
            pltpu.VMEM((L,), jnp.float32),
            pltpu.VMEM((L,), jnp.float32),
            pltpu.VMEM((L,), jnp.int32),
            pltpu.SemaphoreType.DMA,
        ],
        compiler_params=pltpu.CompilerParams(needs_layout_passes=False),
    )
    return fn(mask_i32, p_raw)


def _scan_body(ps_ref, hs_ref, y_ref, h_ref):
    k = pl.program_id(1)

    @pl.when(k == 0)
    def _():
        h_ref[...] = jnp.zeros_like(h_ref)

    ps = ps_ref[0, 0, :, :]
    p = jnp.clip(ps, CLIP_LO, CLIP_HI)
    dt = jnp.log(1.0 / (1.0 - p))
    dth = dt.astype(jnp.bfloat16).astype(jnp.float32)
    pb = p.astype(jnp.bfloat16).astype(jnp.float32)

    row = lax.broadcasted_iota(jnp.int32, (T, T), 0)
    col = lax.broadcasted_iota(jnp.int32, (T, T), 1)
    dt_bc = jnp.broadcast_to(dth, (T, T))
    s_col = jnp.sum(jnp.where(col <= row, dt_bc, 0.0), axis=1, keepdims=True)
    dt_col = jnp.sum(jnp.where(col == row, dt_bc, 0.0), axis=1, keepdims=True)
    tri_u = (row <= col).astype(jnp.float32)
    s_row = jnp.dot(dth, tri_u, preferred_element_type=jnp.float32,
                    precision=lax.Precision.HIGHEST)

    mlog = jnp.broadcast_to(s_row, (T, T)) - s_col
    w = jnp.where(col <= row, jnp.exp(mlog), 0.0) * jnp.broadcast_to(dth * pb, (T, T))

    xs = hs_ref[0, :, :]
    xb = (xs / dt_col).astype(jnp.bfloat16).astype(jnp.float32)

    y0 = jnp.dot(w, xb, preferred_element_type=jnp.float32,
                 precision=lax.Precision.HIGHEST)
    y = y0 + jnp.exp(-s_col) * h_ref[0:1, :]
    y_ref[0, :, :] = y.astype(jnp.bfloat16).astype(jnp.float32)
    h_ref[0:1, :] = y[T - 1:T, :]


def _tc_scan(p_sorted, hidden):
    ps_r = p_sorted.reshape(B, K, 1, T)
    return pl.pallas_call(
        _scan_body,
        grid=(B, K),
        in_specs=[
            pl.BlockSpec((1, 1, 1, T), lambda b, k: (b, k, 0, 0)),
            pl.BlockSpec((1, T, D), lambda b, k: (b, k, 0)),
        ],
        out_specs=pl.BlockSpec((1, T, D), lambda b, k: (b, k, 0)),
        out_shape=jax.ShapeDtypeStruct((B, L, D), jnp.float32),
        scratch_shapes=[pltpu.VMEM((8, D), jnp.float32)],
        compiler_params=pltpu.CompilerParams(
            dimension_semantics=("arbitrary", "arbitrary"),
        ),
    )(ps_r, hidden)


def _gather_body(table_hbm, idx_hbm, out_hbm, idx_v, rows_v, sem):
    wid = lax.axis_index("s") * 2 + lax.axis_index("c")
    base = wid * ROWS_PER_W
    pltpu.async_copy(idx_hbm.at[pl.ds(base, ROWS_PER_W)], idx_v, sem).wait()

    def body(g, carry):
        r0 = g * GCHUNK
        pltpu.async_copy(table_hbm.at[idx_v.at[pl.ds(r0, GCHUNK)]],
                         rows_v, sem).wait()
        pltpu.async_copy(rows_v, out_hbm.at[pl.ds(base + r0, GCHUNK)],
                         sem).wait()
        return carry

    lax.fori_loop(0, ROWS_PER_W // GCHUNK, body, jnp.int32(0))


def _sc_gather(y2d, idx_flat):
    mesh = plsc.VectorSubcoreMesh(core_axis_name="c", subcore_axis_name="s")
    fn = pl.kernel(
        _gather_body,
        mesh=mesh,
        out_type=jax.ShapeDtypeStruct((B * L, D), jnp.float32),
        scratch_types=[
            pltpu.VMEM((ROWS_PER_W,), jnp.int32),
            pltpu.VMEM((GCHUNK, D), jnp.float32),
            pltpu.SemaphoreType.DMA,
        ],
        compiler_params=pltpu.CompilerParams(needs_layout_passes=False),
    )
    return fn(y2d, idx_flat)


def kernel(hidden_states, boundary_mask, boundary_prob):
    mask_i32 = boundary_mask.astype(jnp.int32)
    p_raw = boundary_prob[..., 1].astype(jnp.float32)
    p_sorted, idx_flat = _sc_prep(mask_i32, p_raw)
    y = _tc_scan(p_sorted, hidden_states)
    out = _sc_gather(y.reshape(B * L, D), idx_flat.reshape(B * L))
    return out.reshape(B, L, D)


# speedup vs baseline: 38.8706x; 1.1396x over previous
import functools

import jax
import jax.numpy as jnp
from jax import lax
from jax.experimental import pallas as pl
from jax.experimental.pallas import tpu as pltpu
from jax.experimental.pallas import tpu_sc as plsc

B = 8
L = 4096
D = 1024
T = 256
K = L // T
NW = 32
ROWS_PER_W = (B * L) // NW
GCHUNK = 32
CLIP_LO = 1e-4
CLIP_HI = 1.0 - 1e-4


def _prep_body(mask_hbm, p_hbm, psort_hbm, idx_hbm,
               mask_v, p_v, psort_v, idx_v, sem):
    wid = lax.axis_index("s") * 2 + lax.axis_index("c")

    @pl.when(wid < B)
    def _():
        pltpu.async_copy(mask_hbm.at[wid], mask_v, sem).wait()
        pltpu.async_copy(p_hbm.at[wid], p_v, sem).wait()

        fill = jnp.full((16,), 0.5, jnp.float32)

        def body(j, cnt_vec):
            m16 = mask_v[pl.ds(j * 16, 16)]
            mb = m16 > 0
            p16 = p_v[pl.ds(j * 16, 16)]
            psort_v[pl.ds(j * 16, 16)] = fill
            cum = plsc.cumsum(m16)
            pbv = cum + cnt_vec - 1
            idx_v[pl.ds(j * 16, 16)] = pbv + wid * L
            plsc.store_scatter(psort_v, [pbv], p16, mask=mb)
            return cnt_vec + plsc.all_reduce_population_count(mb)

        lax.fori_loop(0, L // 16, body, jnp.zeros((16,), jnp.int32))

        pltpu.async_copy(psort_v, psort_hbm.at[wid], sem).wait()
        pltpu.async_copy(idx_v, idx_hbm.at[wid], sem).wait()


def _sc_prep(mask_i32, p_raw):
    mesh = plsc.VectorSubcoreMesh(core_axis_name="c", subcore_axis_name="s")
    fn = pl.kernel(
        _prep_body,
        mesh=mesh,
        out_type=(
            jax.ShapeDtypeStruct((B, L), jnp.float32),
            jax.ShapeDtypeStruct((B, L), jnp.int32),
        ),
        scratch_types=[
            pltpu.VMEM((L,), jnp.int32),
            pltpu.VMEM((L,), jnp.float32),
            pltpu.VMEM((L,), jnp.float32),
            pltpu.VMEM((L,), jnp.int32),
            pltpu.SemaphoreType.DMA,
        ],
        compiler_params=pltpu.CompilerParams(needs_layout_passes=False),
    )
    return fn(mask_i32, p_raw)


def _scan_body(ps_ref, hs_ref, y_ref, h_ref):
    k = pl.program_id(1)

    @pl.when(k == 0)
    def _():
        h_ref[...] = jnp.zeros_like(h_ref)

    ps = ps_ref[0, 0, :, :]
    p = jnp.clip(ps, CLIP_LO, CLIP_HI)
    dt = jnp.log(1.0 / (1.0 - p))
    dth = dt.astype(jnp.bfloat16).astype(jnp.float32)
    pb = p.astype(jnp.bfloat16).astype(jnp.float32)

    row = lax.broadcasted_iota(jnp.int32, (T, T), 0)
    col = lax.broadcasted_iota(jnp.int32, (T, T), 1)
    dt_bc = jnp.broadcast_to(dth, (T, T))
    s_col = jnp.sum(jnp.where(col <= row, dt_bc, 0.0), axis=1, keepdims=True)
    dt_col = jnp.sum(jnp.where(col == row, dt_bc, 0.0), axis=1, keepdims=True)
    tri_u = (row <= col).astype(jnp.bfloat16)
    s_row = jnp.dot(dth.astype(jnp.bfloat16), tri_u,
                    preferred_element_type=jnp.float32)

    mlog = jnp.broadcast_to(s_row, (T, T)) - s_col
    w = jnp.where(col <= row, jnp.exp(mlog), 0.0) * jnp.broadcast_to(dth * pb, (T, T))

    xs = hs_ref[0, :, :]
    xb = (xs / dt_col).astype(jnp.bfloat16)

    w_hi = w.astype(jnp.bfloat16)
    w_lo = (w - w_hi.astype(jnp.float32)).astype(jnp.bfloat16)
    y0 = (jnp.dot(w_hi, xb, preferred_element_type=jnp.float32)
          + jnp.dot(w_lo, xb, preferred_element_type=jnp.float32))
    y = y0 + jnp.exp(-s_col) * h_ref[0:1, :]
    y_ref[0, :, :] = y.astype(jnp.bfloat16).astype(jnp.float32)
    h_ref[0:1, :] = y[T - 1:T, :]


def _tc_scan(p_sorted, hidden):
    ps_r = p_sorted.reshape(B, K, 1, T)
    return pl.pallas_call(
        _scan_body,
        grid=(B, K),
        in_specs=[
            pl.BlockSpec((1, 1, 1, T), lambda b, k: (b, k, 0, 0)),
            pl.BlockSpec((1, T, D), lambda b, k: (b, k, 0)),
        ],
        out_specs=pl.BlockSpec((1, T, D), lambda b, k: (b, k, 0)),
        out_shape=jax.ShapeDtypeStruct((B, L, D), jnp.float32),
        scratch_shapes=[pltpu.VMEM((8, D), jnp.float32)],
        compiler_params=pltpu.CompilerParams(
            dimension_semantics=("arbitrary", "arbitrary"),
        ),
    )(ps_r, hidden)


def _gather_body(table_hbm, idx_hbm, out_hbm, idx_v, buf_a, buf_b,
                 sem_ga, sem_gb, sem_sa, sem_sb):
    wid = lax.axis_index("s") * 2 + lax.axis_index("c")
    base = wid * ROWS_PER_W
    pltpu.async_copy(idx_hbm.at[pl.ds(base, ROWS_PER_W)], idx_v, sem_ga).wait()

    n = ROWS_PER_W // GCHUNK
    bufs = (buf_a, buf_b)
    gsems = (sem_ga, sem_gb)
    ssems = (sem_sa, sem_sb)
    gh = [None] * n
    sh = [None] * n

    def start_gather(g):
        return pltpu.async_copy(
            table_hbm.at[idx_v.at[pl.ds(g * GCHUNK, GCHUNK)]],
            bufs[g & 1], gsems[g & 1])

    gh[0] = start_gather(0)
    for g in range(n):
        gh[g].wait()
        if g + 1 < n:
            if g - 1 >= 0:
                sh[g - 1].wait()
            gh[g + 1] = start_gather(g + 1)
        sh[g] = pltpu.async_copy(
            bufs[g & 1], out_hbm.at[pl.ds(base + g * GCHUNK, GCHUNK)],
            ssems[g & 1])
    sh[n - 2].wait()
    sh[n - 1].wait()


def _sc_gather(y2d, idx_flat):
    mesh = plsc.VectorSubcoreMesh(core_axis_name="c", subcore_axis_name="s")
    fn = pl.kernel(
        _gather_body,
        mesh=mesh,
        out_type=jax.ShapeDtypeStruct((B * L, D), jnp.float32),
        scratch_types=[
            pltpu.VMEM((ROWS_PER_W,), jnp.int32),
            pltpu.VMEM((GCHUNK, D), jnp.float32),
            pltpu.VMEM((GCHUNK, D), jnp.float32),
            pltpu.SemaphoreType.DMA,
            pltpu.SemaphoreType.DMA,
            pltpu.SemaphoreType.DMA,
            pltpu.SemaphoreType.DMA,
        ],
        compiler_params=pltpu.CompilerParams(needs_layout_passes=False),
    )
    return fn(y2d, idx_flat)


def kernel(hidden_states, boundary_mask, boundary_prob):
    mask_i32 = boundary_mask.astype(jnp.int32)
    p_raw = boundary_prob[..., 1].astype(jnp.float32)
    p_sorted, idx_flat = _sc_prep(mask_i32, p_raw)
    y = _tc_scan(p_sorted, hidden_states)
    out = _sc_gather(y.reshape(B * L, D), idx_flat.reshape(B * L))
    return out.reshape(B, L, D)


# speedup vs baseline: 48.7117x; 1.2532x over previous
import jax
import jax.numpy as jnp
from jax import lax
from jax.experimental import pallas as pl
from jax.experimental.pallas import tpu as pltpu
from jax.experimental.pallas import tpu_sc as plsc

B = 8
L = 4096
D = 1024
T = 256
K = L // T
WIN = 512
CLIP_LO = 1e-4
CLIP_HI = 1.0 - 1e-4


def _prep_body(mask_hbm, p_hbm, psort_hbm, idx_hbm,
               mask_v, p_v, psort_v, idx_v, sem):
    wid = lax.axis_index("s") * 2 + lax.axis_index("c")

    @pl.when(wid < B)
    def _():
        pltpu.async_copy(mask_hbm.at[wid], mask_v, sem).wait()
        pltpu.async_copy(p_hbm.at[wid], p_v, sem).wait()

        fill = jnp.full((16,), 0.5, jnp.float32)

        def body(j, cnt_vec):
            m16 = mask_v[pl.ds(j * 16, 16)]
            mb = m16 > 0
            p16 = p_v[pl.ds(j * 16, 16)]
            psort_v[pl.ds(j * 16, 16)] = fill
            cum = plsc.cumsum(m16)
            pbv = cum + cnt_vec - 1
            idx_v[pl.ds(j * 16, 16)] = pbv
            plsc.store_scatter(psort_v, [pbv], p16, mask=mb)
            return cnt_vec + plsc.all_reduce_population_count(mb)

        lax.fori_loop(0, L // 16, body, jnp.zeros((16,), jnp.int32))

        pltpu.async_copy(psort_v, psort_hbm.at[wid], sem).wait()
        pltpu.async_copy(idx_v, idx_hbm.at[wid], sem).wait()


def _sc_prep(mask_i32, p_raw):
    mesh = plsc.VectorSubcoreMesh(core_axis_name="c", subcore_axis_name="s")
    fn = pl.kernel(
        _prep_body,
        mesh=mesh,
        out_type=(
            jax.ShapeDtypeStruct((B, L), jnp.float32),
            jax.ShapeDtypeStruct((B, L), jnp.int32),
        ),
        scratch_types=[
            pltpu.VMEM((L,), jnp.int32),
            pltpu.VMEM((L,), jnp.float32),
            pltpu.VMEM((L,), jnp.float32),
            pltpu.VMEM((L,), jnp.int32),
            pltpu.SemaphoreType.DMA,
        ],
        compiler_params=pltpu.CompilerParams(needs_layout_passes=False),
    )
    return fn(mask_i32, p_raw)


def _scan_body(ps_ref, hs_ref, idx_ref, out_ref, h_ref, yscr_ref):
    k = pl.program_id(1)

    @pl.when(k == 0)
    def _():
        h_ref[...] = jnp.zeros_like(h_ref)
        yscr_ref[pl.ds(T, T), :] = jnp.zeros((T, D), jnp.bfloat16)
        yscr_ref[pl.ds(2 * T, T), :] = jnp.zeros((T, D), jnp.bfloat16)

    @pl.when((k > 0) & (k + 2 < K))
    def _():
        yscr_ref[pl.ds((k + 2) * T, T), :] = jnp.zeros((T, D), jnp.bfloat16)

    ps = ps_ref[0, 0, :, :]
    p = jnp.clip(ps, CLIP_LO, CLIP_HI)
    dt = jnp.log(1.0 / (1.0 - p))
    dth = dt.astype(jnp.bfloat16).astype(jnp.float32)
    pb = p.astype(jnp.bfloat16).astype(jnp.float32)

    row = lax.broadcasted_iota(jnp.int32, (T, T), 0)
    col = lax.broadcasted_iota(jnp.int32, (T, T), 1)
    dt_bc = jnp.broadcast_to(dth, (T, T))
    s_col = jnp.sum(jnp.where(col <= row, dt_bc, 0.0), axis=1, keepdims=True)
    dt_col = jnp.sum(jnp.where(col == row, dt_bc, 0.0), axis=1, keepdims=True)
    tri_u = (row <= col).astype(jnp.bfloat16)
    s_row = jnp.dot(dth.astype(jnp.bfloat16), tri_u,
                    preferred_element_type=jnp.float32)

    mlog = jnp.broadcast_to(s_row, (T, T)) - s_col
    w = jnp.where(col <= row, jnp.exp(mlog), 0.0) * jnp.broadcast_to(dth * pb, (T, T))

    xs = hs_ref[0, :, :]
    xb = (xs / dt_col).astype(jnp.bfloat16)

    w_hi = w.astype(jnp.bfloat16)
    w_lo = (w - w_hi.astype(jnp.float32)).astype(jnp.bfloat16)
    y0 = (jnp.dot(w_hi, xb, preferred_element_type=jnp.float32)
          + jnp.dot(w_lo, xb, preferred_element_type=jnp.float32))
    y = y0 + jnp.exp(-s_col) * h_ref[0:1, :]
    h_ref[0:1, :] = y[T - 1:T, :]
    yscr_ref[pl.ds(k * T, T), :] = y.astype(jnp.bfloat16)

    pbcol = idx_ref[0, 0, :, :]
    w0 = idx_ref[0, 0, 0, 0]
    w0a = pl.multiple_of(jnp.minimum((w0 // 16) * 16, L - WIN), 16)
    ywin = yscr_ref[pl.ds(w0a, WIN), :]
    local = pbcol - w0a
    oh = (jnp.broadcast_to(local, (T, WIN))
          == lax.broadcasted_iota(jnp.int32, (T, WIN), 1)).astype(jnp.bfloat16)
    out_ref[0, :, :] = jnp.dot(oh, ywin, preferred_element_type=jnp.float32)


def _tc_scan_gather(p_sorted, hidden, idx):
    ps_r = p_sorted.reshape(B, K, 1, T)
    idx_r = idx.reshape(B, K, T, 1)
    return pl.pallas_call(
        _scan_body,
        grid=(B, K),
        in_specs=[
            pl.BlockSpec((1, 1, 1, T), lambda b, k: (b, k, 0, 0)),
            pl.BlockSpec((1, T, D), lambda b, k: (b, k, 0)),
            pl.BlockSpec((1, 1, T, 1), lambda b, k: (b, k, 0, 0)),
        ],
        out_specs=pl.BlockSpec((1, T, D), lambda b, k: (b, k, 0)),
        out_shape=jax.ShapeDtypeStruct((B, L, D), jnp.float32),
        scratch_shapes=[
            pltpu.VMEM((8, D), jnp.float32),
            pltpu.VMEM((L, D), jnp.bfloat16),
        ],
        compiler_params=pltpu.CompilerParams(
            dimension_semantics=("arbitrary", "arbitrary"),
        ),
    )(ps_r, hidden, idx_r)


def kernel(hidden_states, boundary_mask, boundary_prob):
    mask_i32 = boundary_mask.astype(jnp.int32)
    p_raw = boundary_prob[..., 1].astype(jnp.float32)
    p_sorted, idx = _sc_prep(mask_i32, p_raw)
    return _tc_scan_gather(p_sorted, hidden_states, idx)


# speedup vs baseline: 49.1474x; 1.0089x over previous
import jax
import jax.numpy as jnp
from jax import lax
from jax.experimental import pallas as pl
from jax.experimental.pallas import tpu as pltpu
from jax.experimental.pallas import tpu_sc as plsc

B = 8
L = 4096
D = 1024
T = 256
K = L // T
WIN = 272
CLIP_LO = 1e-4
CLIP_HI = 1.0 - 1e-4


def _prep_body(mask_hbm, p_hbm, psort_hbm, idx_hbm,
               mask_v, p_v, psort_v, idx_v, sem):
    wid = lax.axis_index("s") * 2 + lax.axis_index("c")

    @pl.when(wid < B)
    def _():
        pltpu.async_copy(mask_hbm.at[wid], mask_v, sem).wait()
        pltpu.async_copy(p_hbm.at[wid], p_v, sem).wait()

        fill = jnp.full((16,), 0.5, jnp.float32)

        def body(j, cnt_vec):
            m16 = mask_v[pl.ds(j * 16, 16)]
            mb = m16 > 0
            p16 = p_v[pl.ds(j * 16, 16)]
            psort_v[pl.ds(j * 16, 16)] = fill
            cum = plsc.cumsum(m16)
            pbv = cum + cnt_vec - 1
            idx_v[pl.ds(j * 16, 16)] = pbv
            plsc.store_scatter(psort_v, [pbv], p16, mask=mb)
            return cnt_vec + plsc.all_reduce_population_count(mb)

        lax.fori_loop(0, L // 16, body, jnp.zeros((16,), jnp.int32))

        pltpu.async_copy(psort_v, psort_hbm.at[wid], sem).wait()
        pltpu.async_copy(idx_v, idx_hbm.at[wid], sem).wait()


def _sc_prep(mask_i32, p_raw):
    mesh = plsc.VectorSubcoreMesh(core_axis_name="c", subcore_axis_name="s")
    fn = pl.kernel(
        _prep_body,
        mesh=mesh,
        out_type=(
            jax.ShapeDtypeStruct((B, L), jnp.float32),
            jax.ShapeDtypeStruct((B, L), jnp.int32),
        ),
        scratch_types=[
            pltpu.VMEM((L,), jnp.int32),
            pltpu.VMEM((L,), jnp.float32),
            pltpu.VMEM((L,), jnp.float32),
            pltpu.VMEM((L,), jnp.int32),
            pltpu.SemaphoreType.DMA,
        ],
        compiler_params=pltpu.CompilerParams(needs_layout_passes=False),
    )
    return fn(mask_i32, p_raw)


def _scan_body(ps_ref, hs_ref, idx_ref, out_ref, h_ref, yscr_ref):
    k = pl.program_id(1)

    @pl.when(k == 0)
    def _():
        h_ref[...] = jnp.zeros_like(h_ref)
        yscr_ref[pl.ds(T, WIN - T), :] = jnp.zeros((WIN - T, D), jnp.bfloat16)

    ps = ps_ref[0, 0, :, :]
    p = jnp.clip(ps, CLIP_LO, CLIP_HI)
    dt = jnp.log(1.0 / (1.0 - p))
    dth = dt.astype(jnp.bfloat16).astype(jnp.float32)
    pb = p.astype(jnp.bfloat16).astype(jnp.float32)

    row = lax.broadcasted_iota(jnp.int32, (T, T), 0)
    col = lax.broadcasted_iota(jnp.int32, (T, T), 1)
    dt_bc = jnp.broadcast_to(dth, (T, T))
    s_col = jnp.sum(jnp.where(col <= row, dt_bc, 0.0), axis=1, keepdims=True)
    dt_col = jnp.sum(jnp.where(col == row, dt_bc, 0.0), axis=1, keepdims=True)
    tri_u = (row <= col).astype(jnp.bfloat16)
    s_row = jnp.dot(dth.astype(jnp.bfloat16), tri_u,
                    preferred_element_type=jnp.float32)

    mlog = jnp.broadcast_to(s_row, (T, T)) - s_col
    w = jnp.where(col <= row, jnp.exp(mlog), 0.0) * jnp.broadcast_to(dth * pb, (T, T))

    xs = hs_ref[0, :, :]
    xb = (xs / dt_col).astype(jnp.bfloat16)

    w_hi = w.astype(jnp.bfloat16)
    w_lo = (w - w_hi.astype(jnp.float32)).astype(jnp.bfloat16)
    y0 = (jnp.dot(w_hi, xb, preferred_element_type=jnp.float32)
          + jnp.dot(w_lo, xb, preferred_element_type=jnp.float32))
    y = y0 + jnp.exp(-s_col) * h_ref[0:1, :]
    h_ref[0:1, :] = y[T - 1:T, :]
    yscr_ref[pl.ds(k * T, T), :] = y.astype(jnp.bfloat16)

    pbcol = idx_ref[0, 0, :, :]
    w0 = idx_ref[0, 0, 0, 0]
    w0a = pl.multiple_of(
        jnp.maximum(jnp.minimum((w0 // 16) * 16, (k + 1) * T - WIN), 0), 16)
    ywin = yscr_ref[pl.ds(w0a, WIN), :]
    local = pbcol - w0a
    oh = (jnp.broadcast_to(local, (T, WIN))
          == lax.broadcasted_iota(jnp.int32, (T, WIN), 1)).astype(jnp.bfloat16)
    out_ref[0, :, :] = jnp.dot(oh, ywin, preferred_element_type=jnp.float32)


def _tc_scan_gather(p_sorted, hidden, idx):
    ps_r = p_sorted.reshape(B, K, 1, T)
    idx_r = idx.reshape(B, K, T, 1)
    return pl.pallas_call(
        _scan_body,
        grid=(B, K),
        in_specs=[
            pl.BlockSpec((1, 1, 1, T), lambda b, k: (b, k, 0, 0)),
            pl.BlockSpec((1, T, D), lambda b, k: (b, k, 0)),
            pl.BlockSpec((1, 1, T, 1), lambda b, k: (b, k, 0, 0)),
        ],
        out_specs=pl.BlockSpec((1, T, D), lambda b, k: (b, k, 0)),
        out_shape=jax.ShapeDtypeStruct((B, L, D), jnp.float32),
        scratch_shapes=[
            pltpu.VMEM((8, D), jnp.float32),
            pltpu.VMEM((L, D), jnp.bfloat16),
        ],
        compiler_params=pltpu.CompilerParams(
            dimension_semantics=("arbitrary", "arbitrary"),
        ),
    )(ps_r, hidden, idx_r)


def kernel(hidden_states, boundary_mask, boundary_prob):
    mask_i32 = boundary_mask.astype(jnp.int32)
    p_raw = boundary_prob[..., 1].astype(jnp.float32)
    p_sorted, idx = _sc_prep(mask_i32, p_raw)
    return _tc_scan_gather(p_sorted, hidden_states, idx)


# speedup vs baseline: 61.5996x; 1.2534x over previous
import jax
import jax.numpy as jnp
from jax import lax
from jax.experimental import pallas as pl
from jax.experimental.pallas import tpu as pltpu
from jax.experimental.pallas import tpu_sc as plsc

B = 8
L = 4096
D = 1024
T = 256
K = L // T
TB = 1024
KO = L // TB
JJ = TB // T
WIN = 272
CLIP_LO = 1e-4
CLIP_HI = 1.0 - 1e-4


def _prep_body(mask_hbm, p_hbm, psort_hbm, idx_hbm, nb_hbm,
               mask_v, p_v, psort_v, idx_v, sem):
    wid = lax.axis_index("s") * 2 + lax.axis_index("c")

    @pl.when(wid < B)
    def _():
        pltpu.async_copy(mask_hbm.at[wid], mask_v, sem).wait()
        pltpu.async_copy(p_hbm.at[wid], p_v, sem).wait()

        fill = jnp.full((16,), 0.5, jnp.float32)

        def body(j, cnt_vec):
            m16 = mask_v[pl.ds(j * 16, 16)]
            mb = m16 > 0
            p16 = p_v[pl.ds(j * 16, 16)]
            psort_v[pl.ds(j * 16, 16)] = fill
            cum = plsc.cumsum(m16)
            pbv = cum + cnt_vec - 1
            idx_v[pl.ds(j * 16, 16)] = pbv
            plsc.store_scatter(psort_v, [pbv], p16, mask=mb)
            return cnt_vec + plsc.all_reduce_population_count(mb)

        total = lax.fori_loop(0, L // 16, body, jnp.zeros((16,), jnp.int32))

        pltpu.async_copy(psort_v, psort_hbm.at[wid], sem).wait()
        pltpu.async_copy(idx_v, idx_hbm.at[wid], sem).wait()
        mask_v[pl.ds(0, 16)] = total
        pltpu.async_copy(mask_v.at[pl.ds(0, 16)],
                         nb_hbm.at[pl.ds(wid * 16, 16)], sem).wait()


def _sc_prep(mask_i32, p_raw):
    mesh = plsc.VectorSubcoreMesh(core_axis_name="c", subcore_axis_name="s")
    fn = pl.kernel(
        _prep_body,
        mesh=mesh,
        out_type=(
            jax.ShapeDtypeStruct((B, L), jnp.float32),
            jax.ShapeDtypeStruct((B, L), jnp.int32),
            jax.ShapeDtypeStruct((B * 16,), jnp.int32),
        ),
        scratch_types=[
            pltpu.VMEM((L,), jnp.int32),
            pltpu.VMEM((L,), jnp.float32),
            pltpu.VMEM((L,), jnp.float32),
            pltpu.VMEM((L,), jnp.int32),
            pltpu.SemaphoreType.DMA,
        ],
        compiler_params=pltpu.CompilerParams(needs_layout_passes=False),
    )
    return fn(mask_i32, p_raw)


def _scan_body(nb_ref, ps_ref, hs_ref, idx_ref, out_ref, h_ref, yscr_ref):
    b = pl.program_id(0)
    ko = pl.program_id(1)
    nb = nb_ref[b]

    @pl.when(ko == 0)
    def _():
        h_ref[...] = jnp.zeros_like(h_ref)
        yscr_ref[pl.ds(T, WIN - T), :] = jnp.zeros((WIN - T, D), jnp.bfloat16)

    row = lax.broadcasted_iota(jnp.int32, (T, T), 0)
    col = lax.broadcasted_iota(jnp.int32, (T, T), 1)
    tri_u = (row <= col).astype(jnp.bfloat16)

    for j in range(JJ):
        kg = ko * JJ + j
        valid = kg * T < nb

        @pl.when(valid)
        def _(j=j, kg=kg):
            ps = ps_ref[0, j, :, :]
            p = jnp.clip(ps, CLIP_LO, CLIP_HI)
            dt = jnp.log(1.0 / (1.0 - p))
            dth = dt.astype(jnp.bfloat16).astype(jnp.float32)
            pb = p.astype(jnp.bfloat16).astype(jnp.float32)

            dt_bc = jnp.broadcast_to(dth, (T, T))
            s_col = jnp.sum(jnp.where(col <= row, dt_bc, 0.0),
                            axis=1, keepdims=True)
            dt_col = jnp.sum(jnp.where(col == row, dt_bc, 0.0),
                             axis=1, keepdims=True)
            s_row = jnp.dot(dth.astype(jnp.bfloat16), tri_u,
                            preferred_element_type=jnp.float32)

            mlog = jnp.broadcast_to(s_row, (T, T)) - s_col
            w = (jnp.where(col <= row, jnp.exp(mlog), 0.0)
                 * jnp.broadcast_to(dth * pb, (T, T)))

            xs = hs_ref[0, pl.ds(j * T, T), :]
            xb = (xs / dt_col).astype(jnp.bfloat16)

            w_hi = w.astype(jnp.bfloat16)
            w_lo = (w - w_hi.astype(jnp.float32)).astype(jnp.bfloat16)
            y0 = (jnp.dot(w_hi, xb, preferred_element_type=jnp.float32)
                  + jnp.dot(w_lo, xb, preferred_element_type=jnp.float32))
            y = y0 + jnp.exp(-s_col) * h_ref[0:1, :]
            h_ref[0:1, :] = y[T - 1:T, :]
            yscr_ref[pl.ds(kg * T, T), :] = y.astype(jnp.bfloat16)

        @pl.when(jnp.logical_not(valid) & ((kg - 2) * T < nb))
        def _(kg=kg):
            yscr_ref[pl.ds(kg * T, T), :] = jnp.zeros((T, D), jnp.bfloat16)

        pbcol = idx_ref[0, j, :, :]
        w0 = idx_ref[0, j, 0, 0]
        w0a = pl.multiple_of(
            jnp.maximum(jnp.minimum((w0 // 16) * 16, (kg + 1) * T - WIN), 0),
            16)
        ywin = yscr_ref[pl.ds(w0a, WIN), :]
        local = pbcol - w0a
        oh = (jnp.broadcast_to(local, (T, WIN))
              == lax.broadcasted_iota(jnp.int32, (T, WIN), 1)
              ).astype(jnp.bfloat16)
        out_ref[0, pl.ds(j * T, T), :] = jnp.dot(
            oh, ywin, preferred_element_type=jnp.float32)


def _tc_scan_gather(p_sorted, hidden, idx, nb):
    ps_r = p_sorted.reshape(B, K, 1, T)
    idx_r = idx.reshape(B, K, T, 1)

    def last_blk(nb_ref, b):
        return jnp.minimum(jnp.maximum(nb_ref[b] - 1, 0) // TB, KO - 1)

    grid_spec = pltpu.PrefetchScalarGridSpec(
        num_scalar_prefetch=1,
        grid=(B, KO),
        in_specs=[
            pl.BlockSpec((1, JJ, 1, T),
                         lambda b, ko, nb_ref:
                         (b, jnp.minimum(ko, last_blk(nb_ref, b)), 0, 0)),
            pl.BlockSpec((1, TB, D),
                         lambda b, ko, nb_ref:
                         (b, jnp.minimum(ko, last_blk(nb_ref, b)), 0)),
            pl.BlockSpec((1, JJ, T, 1),
                         lambda b, ko, nb_ref: (b, ko, 0, 0)),
        ],
        out_specs=pl.BlockSpec((1, TB, D), lambda b, ko, nb_ref: (b, ko, 0)),
        scratch_shapes=[
            pltpu.VMEM((8, D), jnp.float32),
            pltpu.VMEM((L, D), jnp.bfloat16),
        ],
    )
    return pl.pallas_call(
        _scan_body,
        grid_spec=grid_spec,
        out_shape=jax.ShapeDtypeStruct((B, L, D), jnp.float32),
        compiler_params=pltpu.CompilerParams(
            dimension_semantics=("arbitrary", "arbitrary"),
        ),
    )(nb, ps_r, hidden, idx_r)


def kernel(hidden_states, boundary_mask, boundary_prob):
    mask_i32 = boundary_mask.astype(jnp.int32)
    p_raw = boundary_prob[..., 1].astype(jnp.float32)
    p_sorted, idx, nb16 = _sc_prep(mask_i32, p_raw)
    nb = nb16.reshape(B, 16)[:, 0]
    return _tc_scan_gather(p_sorted, hidden_states, idx, nb)


# speedup vs baseline: 62.7307x; 1.0184x over previous
import jax
import jax.numpy as jnp
from jax import lax
from jax.experimental import pallas as pl
from jax.experimental.pallas import tpu as pltpu
from jax.experimental.pallas import tpu_sc as plsc

B = 8
L = 4096
D = 1024
T = 256
K = L // T
TB = 1024
KO = L // TB
JJ = TB // T
WIN = 272
CLIP_LO = 1e-4
CLIP_HI = 1.0 - 1e-4


def _prep_body(mask_hbm, p_hbm, psort_hbm, idx_hbm, nb_hbm,
               mask_v, p_v, psort_v, idx_v, sem):
    wid = lax.axis_index("s") * 2 + lax.axis_index("c")

    @pl.when(wid < B)
    def _():
        pltpu.async_copy(mask_hbm.at[wid], mask_v, sem).wait()
        pltpu.async_copy(p_hbm.at[wid], p_v, sem).wait()

        fill = jnp.full((16,), 0.5, jnp.float32)

        def body(j, cnt_vec):
            m16 = mask_v[pl.ds(j * 16, 16)]
            mb = m16 > 0
            p16 = p_v[pl.ds(j * 16, 16)]
            psort_v[pl.ds(j * 16, 16)] = fill
            cum = plsc.cumsum(m16)
            pbv = cum + cnt_vec - 1
            idx_v[pl.ds(j * 16, 16)] = pbv
            plsc.store_scatter(psort_v, [pbv], p16, mask=mb)
            return cnt_vec + plsc.all_reduce_population_count(mb)

        total = lax.fori_loop(0, L // 16, body, jnp.zeros((16,), jnp.int32))

        pltpu.async_copy(psort_v, psort_hbm.at[wid], sem).wait()
        pltpu.async_copy(idx_v, idx_hbm.at[wid], sem).wait()
        mask_v[pl.ds(0, 16)] = total
        pltpu.async_copy(mask_v.at[pl.ds(0, 16)],
                         nb_hbm.at[pl.ds(wid * 16, 16)], sem).wait()


def _sc_prep(mask_i32, p_raw):
    mesh = plsc.VectorSubcoreMesh(core_axis_name="c", subcore_axis_name="s")
    fn = pl.kernel(
        _prep_body,
        mesh=mesh,
        out_type=(
            jax.ShapeDtypeStruct((B, L), jnp.float32),
            jax.ShapeDtypeStruct((B, L), jnp.int32),
            jax.ShapeDtypeStruct((B * 16,), jnp.int32),
        ),
        scratch_types=[
            pltpu.VMEM((L,), jnp.int32),
            pltpu.VMEM((L,), jnp.float32),
            pltpu.VMEM((L,), jnp.float32),
            pltpu.VMEM((L,), jnp.int32),
            pltpu.SemaphoreType.DMA,
        ],
        compiler_params=pltpu.CompilerParams(needs_layout_passes=False),
    )
    return fn(mask_i32, p_raw)


def _scan_body(nb_ref, ps_ref, hs_ref, idx_ref, out_ref, h_ref, yscr_ref):
    b = pl.program_id(0)
    ko = pl.program_id(1)
    nb = nb_ref[b]

    @pl.when(ko == 0)
    def _():
        h_ref[...] = jnp.zeros_like(h_ref)
        yscr_ref[pl.ds(T, WIN - T), :] = jnp.zeros((WIN - T, D), jnp.bfloat16)

    row = lax.broadcasted_iota(jnp.int32, (T, T), 0)
    col = lax.broadcasted_iota(jnp.int32, (T, T), 1)
    tri_u = (row <= col).astype(jnp.bfloat16)

    for j in range(JJ):
        kg = ko * JJ + j
        valid = kg * T < nb

        @pl.when(valid)
        def _(j=j, kg=kg):
            ps = ps_ref[0, j, :, :]
            p = jnp.clip(ps, CLIP_LO, CLIP_HI)
            dt = jnp.log(1.0 / (1.0 - p))
            dth = dt.astype(jnp.bfloat16).astype(jnp.float32)
            pb = p.astype(jnp.bfloat16).astype(jnp.float32)

            dt_bc = jnp.broadcast_to(dth, (T, T))
            s_col = jnp.sum(jnp.where(col <= row, dt_bc, 0.0),
                            axis=1, keepdims=True)
            s_row = jnp.dot(dth.astype(jnp.bfloat16), tri_u,
                            preferred_element_type=jnp.float32)

            mlog = jnp.broadcast_to(s_row, (T, T)) - s_col
            w = (jnp.where(col <= row, jnp.exp(mlog), 0.0)
                 * jnp.broadcast_to(pb, (T, T)))

            xs = hs_ref[0, pl.ds(j * T, T), :]
            xb = xs.astype(jnp.bfloat16)

            w_hi = w.astype(jnp.bfloat16)
            w_lo = (w - w_hi.astype(jnp.float32)).astype(jnp.bfloat16)
            y0 = (jnp.dot(w_hi, xb, preferred_element_type=jnp.float32)
                  + jnp.dot(w_lo, xb, preferred_element_type=jnp.float32))
            y = y0 + jnp.exp(-s_col) * h_ref[0:1, :]
            h_ref[0:1, :] = y[T - 1:T, :]
            yscr_ref[pl.ds(kg * T, T), :] = y.astype(jnp.bfloat16)

        @pl.when(jnp.logical_not(valid) & ((kg - 2) * T < nb))
        def _(kg=kg):
            yscr_ref[pl.ds(kg * T, T), :] = jnp.zeros((T, D), jnp.bfloat16)

        pbcol = idx_ref[0, j, :, :]
        w0 = idx_ref[0, j, 0, 0]
        w0a = pl.multiple_of(
            jnp.maximum(jnp.minimum((w0 // 16) * 16, (kg + 1) * T - WIN), 0),
            16)
        ywin = yscr_ref[pl.ds(w0a, WIN), :]
        local = pbcol - w0a
        oh = (jnp.broadcast_to(local, (T, WIN))
              == lax.broadcasted_iota(jnp.int32, (T, WIN), 1)
              ).astype(jnp.bfloat16)
        out_ref[0, pl.ds(j * T, T), :] = jnp.dot(
            oh, ywin, preferred_element_type=jnp.float32)


def _tc_scan_gather(p_sorted, hidden, idx, nb):
    ps_r = p_sorted.reshape(B, K, 1, T)
    idx_r = idx.reshape(B, K, T, 1)

    def last_blk(nb_ref, b):
        return jnp.minimum(jnp.maximum(nb_ref[b] - 1, 0) // TB, KO - 1)

    grid_spec = pltpu.PrefetchScalarGridSpec(
        num_scalar_prefetch=1,
        grid=(B, KO),
        in_specs=[
            pl.BlockSpec((1, JJ, 1, T),
                         lambda b, ko, nb_ref:
                         (b, jnp.minimum(ko, last_blk(nb_ref, b)), 0, 0)),
            pl.BlockSpec((1, TB, D),
                         lambda b, ko, nb_ref:
                         (b, jnp.minimum(ko, last_blk(nb_ref, b)), 0)),
            pl.BlockSpec((1, JJ, T, 1),
                         lambda b, ko, nb_ref: (b, ko, 0, 0)),
        ],
        out_specs=pl.BlockSpec((1, TB, D), lambda b, ko, nb_ref: (b, ko, 0)),
        scratch_shapes=[
            pltpu.VMEM((8, D), jnp.float32),
            pltpu.VMEM((L, D), jnp.bfloat16),
        ],
    )
    return pl.pallas_call(
        _scan_body,
        grid_spec=grid_spec,
        out_shape=jax.ShapeDtypeStruct((B, L, D), jnp.float32),
        compiler_params=pltpu.CompilerParams(
            dimension_semantics=("arbitrary", "arbitrary"),
        ),
    )(nb, ps_r, hidden, idx_r)


def kernel(hidden_states, boundary_mask, boundary_prob):
    mask_i32 = boundary_mask.astype(jnp.int32)
    p_raw = boundary_prob[..., 1].astype(jnp.float32)
    p_sorted, idx, nb16 = _sc_prep(mask_i32, p_raw)
    nb = nb16.reshape(B, 16)[:, 0]
    return _tc_scan_gather(p_sorted, hidden_states, idx, nb)


# speedup vs baseline: 66.1417x; 1.0544x over previous
import jax
import jax.numpy as jnp
from jax import lax
from jax.experimental import pallas as pl
from jax.experimental.pallas import tpu as pltpu
from jax.experimental.pallas import tpu_sc as plsc

B = 8
L = 4096
D = 1024
T = 256
K = L // T
TB = 1024
KO = L // TB
JJ = TB // T
WIN = 272
CLIP_LO = 1e-4
CLIP_HI = 1.0 - 1e-4


def _prep_body(mask_hbm, p_hbm, psort_hbm, idx_hbm, nb_hbm,
               mask_v, p_v, psort_v, idx_v, sem):
    wid = lax.axis_index("s") * 2 + lax.axis_index("c")

    @pl.when(wid < B)
    def _():
        pltpu.async_copy(mask_hbm.at[wid], mask_v, sem).wait()
        pltpu.async_copy(p_hbm.at[wid], p_v, sem).wait()

        fill = jnp.full((16,), 0.5, jnp.float32)

        def body(j, cnt_vec):
            m16 = mask_v[pl.ds(j * 16, 16)]
            mb = m16 > 0
            p16 = p_v[pl.ds(j * 16, 16)]
            psort_v[pl.ds(j * 16, 16)] = fill
            cum = plsc.cumsum(m16)
            pbv = cum + cnt_vec - 1
            idx_v[pl.ds(j * 16, 16)] = pbv
            plsc.store_scatter(psort_v, [pbv], p16, mask=mb)
            return cnt_vec + plsc.all_reduce_population_count(mb)

        total = lax.fori_loop(0, L // 16, body, jnp.zeros((16,), jnp.int32))

        pltpu.async_copy(psort_v, psort_hbm.at[wid], sem).wait()
        pltpu.async_copy(idx_v, idx_hbm.at[wid], sem).wait()
        mask_v[pl.ds(0, 16)] = total
        pltpu.async_copy(mask_v.at[pl.ds(0, 16)],
                         nb_hbm.at[pl.ds(wid * 16, 16)], sem).wait()


def _sc_prep(mask_i32, p_raw):
    mesh = plsc.VectorSubcoreMesh(core_axis_name="c", subcore_axis_name="s")
    fn = pl.kernel(
        _prep_body,
        mesh=mesh,
        out_type=(
            jax.ShapeDtypeStruct((B, L), jnp.float32),
            jax.ShapeDtypeStruct((B, L), jnp.int32),
            jax.ShapeDtypeStruct((B * 16,), jnp.int32),
        ),
        scratch_types=[
            pltpu.VMEM((L,), jnp.int32),
            pltpu.VMEM((L,), jnp.float32),
            pltpu.VMEM((L,), jnp.float32),
            pltpu.VMEM((L,), jnp.int32),
            pltpu.SemaphoreType.DMA,
        ],
        compiler_params=pltpu.CompilerParams(needs_layout_passes=False),
    )
    return fn(mask_i32, p_raw)


def _scan_body(nb_ref, ps_ref, hs_ref, idx_ref, out_ref, h_ref, yscr_ref):
    b = pl.program_id(0)
    ko = pl.program_id(1)
    nb = nb_ref[b]

    @pl.when(ko == 0)
    def _():
        h_ref[...] = jnp.zeros_like(h_ref)
        yscr_ref[pl.ds(T, WIN - T), :] = jnp.zeros((WIN - T, D), jnp.bfloat16)

    row = lax.broadcasted_iota(jnp.int32, (T, T), 0)
    col = lax.broadcasted_iota(jnp.int32, (T, T), 1)
    tri_u = (row <= col).astype(jnp.bfloat16)

    for j in range(JJ):
        kg = ko * JJ + j
        valid = kg * T < nb

        @pl.when(valid)
        def _(j=j, kg=kg):
            ps = ps_ref[0, j, :, :]
            p = jnp.clip(ps, CLIP_LO, CLIP_HI)
            dt = jnp.log(1.0 / (1.0 - p))
            dth = dt.astype(jnp.bfloat16).astype(jnp.float32)
            pb = p.astype(jnp.bfloat16).astype(jnp.float32)

            dt_bc = jnp.broadcast_to(dth, (T, T))
            s_col = jnp.sum(jnp.where(col <= row, dt_bc, 0.0),
                            axis=1, keepdims=True)
            s_row = jnp.dot(dth.astype(jnp.bfloat16), tri_u,
                            preferred_element_type=jnp.float32)

            mlog = jnp.broadcast_to(s_row, (T, T)) - s_col
            w = (jnp.where(col <= row, jnp.exp(mlog), 0.0)
                 * jnp.broadcast_to(pb, (T, T)))

            xs = hs_ref[0, pl.ds(j * T, T), :]
            xb = xs.astype(jnp.bfloat16)

            y0 = jnp.dot(w.astype(jnp.bfloat16), xb,
                         preferred_element_type=jnp.float32)
            y = y0 + jnp.exp(-s_col) * h_ref[0:1, :]
            h_ref[0:1, :] = y[T - 1:T, :]
            yscr_ref[pl.ds(kg * T, T), :] = y.astype(jnp.bfloat16)

        @pl.when(jnp.logical_not(valid) & ((kg - 2) * T < nb))
        def _(kg=kg):
            yscr_ref[pl.ds(kg * T, T), :] = jnp.zeros((T, D), jnp.bfloat16)

        pbcol = idx_ref[0, j, :, :]
        w0 = idx_ref[0, j, 0, 0]
        w0a = pl.multiple_of(
            jnp.maximum(jnp.minimum((w0 // 16) * 16, (kg + 1) * T - WIN), 0),
            16)
        ywin = yscr_ref[pl.ds(w0a, WIN), :]
        local = pbcol - w0a
        oh = (jnp.broadcast_to(local, (T, WIN))
              == lax.broadcasted_iota(jnp.int32, (T, WIN), 1)
              ).astype(jnp.bfloat16)
        out_ref[0, pl.ds(j * T, T), :] = jnp.dot(
            oh, ywin, preferred_element_type=jnp.float32)


def _tc_scan_gather(p_sorted, hidden, idx, nb):
    ps_r = p_sorted.reshape(B, K, 1, T)
    idx_r = idx.reshape(B, K, T, 1)

    def last_blk(nb_ref, b):
        return jnp.minimum(jnp.maximum(nb_ref[b] - 1, 0) // TB, KO - 1)

    grid_spec = pltpu.PrefetchScalarGridSpec(
        num_scalar_prefetch=1,
        grid=(B, KO),
        in_specs=[
            pl.BlockSpec((1, JJ, 1, T),
                         lambda b, ko, nb_ref:
                         (b, jnp.minimum(ko, last_blk(nb_ref, b)), 0, 0)),
            pl.BlockSpec((1, TB, D),
                         lambda b, ko, nb_ref:
                         (b, jnp.minimum(ko, last_blk(nb_ref, b)), 0)),
            pl.BlockSpec((1, JJ, T, 1),
                         lambda b, ko, nb_ref: (b, ko, 0, 0)),
        ],
        out_specs=pl.BlockSpec((1, TB, D), lambda b, ko, nb_ref: (b, ko, 0)),
        scratch_shapes=[
            pltpu.VMEM((8, D), jnp.float32),
            pltpu.VMEM((L, D), jnp.bfloat16),
        ],
    )
    return pl.pallas_call(
        _scan_body,
        grid_spec=grid_spec,
        out_shape=jax.ShapeDtypeStruct((B, L, D), jnp.float32),
        compiler_params=pltpu.CompilerParams(
            dimension_semantics=("arbitrary", "arbitrary"),
        ),
    )(nb, ps_r, hidden, idx_r)


def kernel(hidden_states, boundary_mask, boundary_prob):
    mask_i32 = boundary_mask.astype(jnp.int32)
    p_raw = boundary_prob[..., 1].astype(jnp.float32)
    p_sorted, idx, nb16 = _sc_prep(mask_i32, p_raw)
    nb = nb16.reshape(B, 16)[:, 0]
    return _tc_scan_gather(p_sorted, hidden_states, idx, nb)
